# baseline (speedup 1.0000x reference)
import functools

import jax
import jax.numpy as jnp
from jax import lax
from jax.experimental import pallas as pl
from jax.experimental.pallas import tpu as pltpu
from jax.experimental.pallas import tpu_sc as plsc

_B, _L, _D = 32, 2048, 1024
_NC, _NS = 2, 16
_CH = _D // 128

_mesh = plsc.VectorSubcoreMesh(core_axis_name="c", subcore_axis_name="s")


@functools.partial(
    pl.kernel,
    mesh=_mesh,
    out_type=jax.ShapeDtypeStruct((_B, _CH, 128), jnp.float32),
    scratch_types=[
        pltpu.VMEM((16,), jnp.int32),
        pltpu.VMEM((_CH, 128), jnp.float32),
        pltpu.VMEM((_CH, 128), jnp.float32),
        pltpu.SemaphoreType.DMA,
        pltpu.SemaphoreType.DMA,
    ],
)
def _pooler(wt_hbm, mut_hbm, idx_hbm, out_hbm, idx_v, wt_v, mut_v, sem1, sem2):
    w = lax.axis_index("s") * _NC + lax.axis_index("c")
    pltpu.sync_copy(idx_hbm.at[w], idx_v)
    idx8 = idx_v.at[pl.ds(0, _CH)]
    cp_wt = pltpu.async_copy(wt_hbm.at[idx8], wt_v, sem1)
    cp_mut = pltpu.async_copy(mut_hbm.at[idx8], mut_v, sem2)
    cp_wt.wait()
    cp_mut.wait()
    for j in range(_CH):
        for k in range(128 // 16):
            sl = pl.ds(k * 16, 16)
            wt_v[j, sl] = wt_v[j, sl] + mut_v[j, sl]
    pltpu.sync_copy(wt_v, out_hbm.at[w])


def kernel(wt_embedding, mut_embedding, positions):
    base = (
        jnp.arange(_B, dtype=jnp.int32) * (_L * _CH)
        + positions.astype(jnp.int32) * _CH
    )
    idx = base[:, None] + jnp.arange(_CH, dtype=jnp.int32)[None, :]
    idx16 = jnp.concatenate([idx, idx], axis=1)
    wt = wt_embedding.reshape(_B * _L * _CH, 128)
    mut = mut_embedding.reshape(_B * _L * _CH, 128)
    out = _pooler(wt, mut, idx16)
    return out.reshape(_B, _D)


# speedup vs baseline: 26.1063x; 26.1063x over previous
import functools

import jax
import jax.numpy as jnp
from jax import lax
from jax.experimental import pallas as pl
from jax.experimental.pallas import tpu as pltpu
from jax.experimental.pallas import tpu_sc as plsc

_B, _L, _D = 32, 2048, 1024
_NC, _NS = 2, 16

_mesh = plsc.VectorSubcoreMesh(core_axis_name="c", subcore_axis_name="s")


@functools.partial(
    pl.kernel,
    mesh=_mesh,
    out_type=jax.ShapeDtypeStruct((_B, _D), jnp.float32),
    scratch_types=[
        pltpu.VMEM((16,), jnp.int32),
        pltpu.VMEM((1, _D), jnp.float32),
        pltpu.VMEM((1, _D), jnp.float32),
        pltpu.SemaphoreType.DMA,
        pltpu.SemaphoreType.DMA,
    ],
)
def _pooler(wt_hbm, mut_hbm, idx_hbm, out_hbm, idx_v, wt_v, mut_v, sem1, sem2):
    w = lax.axis_index("s") * _NC + lax.axis_index("c")
    pltpu.sync_copy(idx_hbm.at[w], idx_v)
    idx1 = idx_v.at[pl.ds(0, 1)]
    cp_wt = pltpu.async_copy(wt_hbm.at[idx1], wt_v, sem1)
    cp_mut = pltpu.async_copy(mut_hbm.at[idx1], mut_v, sem2)
    cp_wt.wait()
    cp_mut.wait()
    for k in range(_D // 16):
        sl = pl.ds(k * 16, 16)
        wt_v[0, sl] = wt_v[0, sl] + mut_v[0, sl]
    pltpu.sync_copy(wt_v, out_hbm.at[pl.ds(w, 1)])


def kernel(wt_embedding, mut_embedding, positions):
    flat = jnp.arange(_B, dtype=jnp.int32) * _L + positions.astype(jnp.int32)
    idx16 = jnp.broadcast_to(flat[:, None], (_B, 16))
    wt = wt_embedding.reshape(_B * _L, _D)
    mut = mut_embedding.reshape(_B * _L, _D)
    return _pooler(wt, mut, idx16)
